# Initial kernel scaffold; baseline (speedup 1.0000x reference)
import jax
import jax.numpy as jnp
from jax.experimental import pallas as pl


def kernel(x, W1, b1, W2, b2):
    raise NotImplementedError("write your pallas kernel here")


# speedup vs baseline: 4.1546x; 4.1546x over previous
import functools

import jax
import jax.numpy as jnp
from jax.experimental import pallas as pl
from jax.experimental.pallas import tpu as pltpu

H = 768
E = 64
K = 2
TM = 512


def _router_body(nsteps, ntokens, x_ref, w1_ref, b1_ref, w2_ref, b2_ref,
                 e_ref, m_ref, u_ref, l_ref):
    step = pl.program_id(0)

    x = x_ref[...]
    h = jnp.dot(x, w1_ref[...], preferred_element_type=jnp.float32)
    h = h + b1_ref[...]
    h = 0.5 * h * (1.0 + jax.lax.erf(h * 0.7071067811865476))
    e = jnp.dot(h, w2_ref[...], preferred_element_type=jnp.float32)
    e = e + b2_ref[...]
    e_ref[...] = e

    lane = jax.lax.broadcasted_iota(jnp.int32, e.shape, 1)
    m1 = jnp.max(e, axis=1, keepdims=True)
    i1 = jnp.min(jnp.where(e == m1, lane, E), axis=1, keepdims=True)
    oh1 = lane == i1
    e2 = jnp.where(oh1, -jnp.inf, e)
    m2 = jnp.max(e2, axis=1, keepdims=True)
    i2 = jnp.min(jnp.where(e2 == m2, lane, E), axis=1, keepdims=True)
    oh2 = lane == i2

    eps = jnp.exp(m2 - m1)
    denom = 1.0 + eps
    w1 = 1.0 / denom
    w2 = eps / denom
    masks = jnp.where(oh1, w1, 0.0) + jnp.where(oh2, w2, 0.0)
    m_ref[...] = masks

    usage_part = jnp.sum(masks, axis=0, keepdims=True)
    cnt_part = jnp.sum((masks > 0.0).astype(jnp.float32))

    @pl.when(step == 0)
    def _init():
        u_ref[...] = jnp.zeros_like(u_ref)
        l_ref[...] = jnp.zeros_like(l_ref)

    u_ref[...] += usage_part
    l_ref[...] += jnp.reshape(cnt_part, (1, 1))

    @pl.when(step == nsteps - 1)
    def _finalize():
        u = u_ref[...]
        un = u / jnp.sum(u)
        u_ref[...] = un
        lbl = jnp.mean((un - 1.0 / E) ** 2)
        sparsity = l_ref[...] / (ntokens * K)
        l_ref[...] = lbl + 0.1 * sparsity


def kernel(x, W1, b1, W2, b2):
    B, S, _ = x.shape
    N = B * S
    nsteps = N // TM
    xf = x.reshape(N, H)

    grid_spec = pl.GridSpec(
        grid=(nsteps,),
        in_specs=[
            pl.BlockSpec((TM, H), lambda i: (i, 0)),
            pl.BlockSpec((H, 2 * H), lambda i: (0, 0)),
            pl.BlockSpec((1, 2 * H), lambda i: (0, 0)),
            pl.BlockSpec((2 * H, E), lambda i: (0, 0)),
            pl.BlockSpec((1, E), lambda i: (0, 0)),
        ],
        out_specs=[
            pl.BlockSpec((TM, E), lambda i: (i, 0)),
            pl.BlockSpec((TM, E), lambda i: (i, 0)),
            pl.BlockSpec((1, E), lambda i: (0, 0)),
            pl.BlockSpec((1, 1), lambda i: (0, 0)),
        ],
    )

    e_out, masks, usage, loss = pl.pallas_call(
        functools.partial(_router_body, nsteps, N),
        grid_spec=grid_spec,
        out_shape=[
            jax.ShapeDtypeStruct((N, E), jnp.float32),
            jax.ShapeDtypeStruct((N, E), jnp.float32),
            jax.ShapeDtypeStruct((1, E), jnp.float32),
            jax.ShapeDtypeStruct((1, 1), jnp.float32),
        ],
        compiler_params=pltpu.CompilerParams(
            dimension_semantics=("arbitrary",),
        ),
    )(xf, W1, b1.reshape(1, 2 * H), W2, b2.reshape(1, E))

    return (e_out.reshape(B, S, E), masks.reshape(B, S, E),
            loss[0, 0], usage[0])


# speedup vs baseline: 4.6588x; 1.1214x over previous
import functools

import jax
import jax.numpy as jnp
from jax.experimental import pallas as pl
from jax.experimental.pallas import tpu as pltpu

H = 768
E = 64
K = 2
TM = 1024


def _router_body(nsteps, ntokens, x_ref, w1_ref, b1_ref, w2_ref, b2_ref,
                 e_ref, m_ref, u_ref, l_ref):
    step = pl.program_id(0)

    x = x_ref[...]
    h = jnp.dot(x, w1_ref[...], preferred_element_type=jnp.float32)
    h = h + b1_ref[...]
    h = 0.5 * h * (1.0 + jax.lax.erf(h * 0.7071067811865476))
    e = jnp.dot(h, w2_ref[...], preferred_element_type=jnp.float32)
    e = e + b2_ref[...]
    e_ref[...] = e

    lane = jax.lax.broadcasted_iota(jnp.int32, e.shape, 1)
    m1 = jnp.max(e, axis=1, keepdims=True)
    i1 = jnp.min(jnp.where(e == m1, lane, E), axis=1, keepdims=True)
    oh1 = lane == i1
    e2 = jnp.where(oh1, -jnp.inf, e)
    m2 = jnp.max(e2, axis=1, keepdims=True)
    i2 = jnp.min(jnp.where(e2 == m2, lane, E), axis=1, keepdims=True)
    oh2 = lane == i2

    eps = jnp.exp(m2 - m1)
    denom = 1.0 + eps
    w1 = 1.0 / denom
    w2 = eps / denom
    masks = jnp.where(oh1, w1, 0.0) + jnp.where(oh2, w2, 0.0)
    m_ref[...] = masks

    usage_part = jnp.sum(masks, axis=0, keepdims=True)
    cnt_part = jnp.sum((masks > 0.0).astype(jnp.float32))

    @pl.when(step == 0)
    def _init():
        u_ref[...] = jnp.zeros_like(u_ref)
        l_ref[...] = jnp.zeros_like(l_ref)

    u_ref[...] += usage_part
    l_ref[...] += jnp.reshape(cnt_part, (1, 1))

    @pl.when(step == nsteps - 1)
    def _finalize():
        u = u_ref[...]
        un = u / jnp.sum(u)
        u_ref[...] = un
        lbl = jnp.mean((un - 1.0 / E) ** 2)
        sparsity = l_ref[...] / (ntokens * K)
        l_ref[...] = lbl + 0.1 * sparsity


def kernel(x, W1, b1, W2, b2):
    B, S, _ = x.shape
    N = B * S
    nsteps = N // TM
    xf = x.reshape(N, H)

    grid_spec = pl.GridSpec(
        grid=(nsteps,),
        in_specs=[
            pl.BlockSpec((TM, H), lambda i: (i, 0)),
            pl.BlockSpec((H, 2 * H), lambda i: (0, 0)),
            pl.BlockSpec((1, 2 * H), lambda i: (0, 0)),
            pl.BlockSpec((2 * H, E), lambda i: (0, 0)),
            pl.BlockSpec((1, E), lambda i: (0, 0)),
        ],
        out_specs=[
            pl.BlockSpec((TM, E), lambda i: (i, 0)),
            pl.BlockSpec((TM, E), lambda i: (i, 0)),
            pl.BlockSpec((1, E), lambda i: (0, 0)),
            pl.BlockSpec((1, 1), lambda i: (0, 0)),
        ],
    )

    e_out, masks, usage, loss = pl.pallas_call(
        functools.partial(_router_body, nsteps, N),
        grid_spec=grid_spec,
        out_shape=[
            jax.ShapeDtypeStruct((N, E), jnp.float32),
            jax.ShapeDtypeStruct((N, E), jnp.float32),
            jax.ShapeDtypeStruct((1, E), jnp.float32),
            jax.ShapeDtypeStruct((1, 1), jnp.float32),
        ],
        compiler_params=pltpu.CompilerParams(
            dimension_semantics=("arbitrary",),
        ),
    )(xf, W1, b1.reshape(1, 2 * H), W2, b2.reshape(1, E))

    return (e_out.reshape(B, S, E), masks.reshape(B, S, E),
            loss[0, 0], usage[0])


# speedup vs baseline: 4.7610x; 1.0219x over previous
import functools

import jax
import jax.numpy as jnp
from jax.experimental import pallas as pl
from jax.experimental.pallas import tpu as pltpu

H = 768
E = 64
K = 2
TM = 2048


def _router_body(nsteps, ntokens, x_ref, w1_ref, b1_ref, w2_ref, b2_ref,
                 e_ref, m_ref, u_ref, l_ref):
    step = pl.program_id(0)

    x = x_ref[...]
    h = jnp.dot(x, w1_ref[...], preferred_element_type=jnp.float32)
    h = h + b1_ref[...]
    h = 0.5 * h * (1.0 + jax.lax.erf(h * 0.7071067811865476))
    e = jnp.dot(h, w2_ref[...], preferred_element_type=jnp.float32)
    e = e + b2_ref[...]
    e_ref[...] = e

    lane = jax.lax.broadcasted_iota(jnp.int32, e.shape, 1)
    m1 = jnp.max(e, axis=1, keepdims=True)
    i1 = jnp.min(jnp.where(e == m1, lane, E), axis=1, keepdims=True)
    oh1 = lane == i1
    e2 = jnp.where(oh1, -jnp.inf, e)
    m2 = jnp.max(e2, axis=1, keepdims=True)
    i2 = jnp.min(jnp.where(e2 == m2, lane, E), axis=1, keepdims=True)
    oh2 = lane == i2

    eps = jnp.exp(m2 - m1)
    denom = 1.0 + eps
    w1 = 1.0 / denom
    w2 = eps / denom
    masks = jnp.where(oh1, w1, 0.0) + jnp.where(oh2, w2, 0.0)
    m_ref[...] = masks

    usage_part = jnp.sum(masks, axis=0, keepdims=True)
    cnt_part = jnp.sum((masks > 0.0).astype(jnp.float32))

    @pl.when(step == 0)
    def _init():
        u_ref[...] = jnp.zeros_like(u_ref)
        l_ref[...] = jnp.zeros_like(l_ref)

    u_ref[...] += usage_part
    l_ref[...] += jnp.reshape(cnt_part, (1, 1))

    @pl.when(step == nsteps - 1)
    def _finalize():
        u = u_ref[...]
        un = u / jnp.sum(u)
        u_ref[...] = un
        lbl = jnp.mean((un - 1.0 / E) ** 2)
        sparsity = l_ref[...] / (ntokens * K)
        l_ref[...] = lbl + 0.1 * sparsity


def kernel(x, W1, b1, W2, b2):
    B, S, _ = x.shape
    N = B * S
    nsteps = N // TM
    xf = x.reshape(N, H)

    grid_spec = pl.GridSpec(
        grid=(nsteps,),
        in_specs=[
            pl.BlockSpec((TM, H), lambda i: (i, 0)),
            pl.BlockSpec((H, 2 * H), lambda i: (0, 0)),
            pl.BlockSpec((1, 2 * H), lambda i: (0, 0)),
            pl.BlockSpec((2 * H, E), lambda i: (0, 0)),
            pl.BlockSpec((1, E), lambda i: (0, 0)),
        ],
        out_specs=[
            pl.BlockSpec((TM, E), lambda i: (i, 0)),
            pl.BlockSpec((TM, E), lambda i: (i, 0)),
            pl.BlockSpec((1, E), lambda i: (0, 0)),
            pl.BlockSpec((1, 1), lambda i: (0, 0)),
        ],
    )

    e_out, masks, usage, loss = pl.pallas_call(
        functools.partial(_router_body, nsteps, N),
        grid_spec=grid_spec,
        out_shape=[
            jax.ShapeDtypeStruct((N, E), jnp.float32),
            jax.ShapeDtypeStruct((N, E), jnp.float32),
            jax.ShapeDtypeStruct((1, E), jnp.float32),
            jax.ShapeDtypeStruct((1, 1), jnp.float32),
        ],
        compiler_params=pltpu.CompilerParams(
            dimension_semantics=("arbitrary",),
        ),
    )(xf, W1, b1.reshape(1, 2 * H), W2, b2.reshape(1, E))

    return (e_out.reshape(B, S, E), masks.reshape(B, S, E),
            loss[0, 0], usage[0])


# speedup vs baseline: 4.8262x; 1.0137x over previous
import functools

import jax
import jax.numpy as jnp
from jax.experimental import pallas as pl
from jax.experimental.pallas import tpu as pltpu

H = 768
E = 64
K = 2
TM = 4096


def _router_body(nsteps, ntokens, x_ref, w1_ref, b1_ref, w2_ref, b2_ref,
                 e_ref, m_ref, u_ref, l_ref):
    step = pl.program_id(0)

    x = x_ref[...]
    h = jnp.dot(x, w1_ref[...], preferred_element_type=jnp.float32)
    h = h + b1_ref[...]
    h = 0.5 * h * (1.0 + jax.lax.erf(h * 0.7071067811865476))
    e = jnp.dot(h, w2_ref[...], preferred_element_type=jnp.float32)
    e = e + b2_ref[...]
    e_ref[...] = e

    lane = jax.lax.broadcasted_iota(jnp.int32, e.shape, 1)
    m1 = jnp.max(e, axis=1, keepdims=True)
    i1 = jnp.min(jnp.where(e == m1, lane, E), axis=1, keepdims=True)
    oh1 = lane == i1
    e2 = jnp.where(oh1, -jnp.inf, e)
    m2 = jnp.max(e2, axis=1, keepdims=True)
    i2 = jnp.min(jnp.where(e2 == m2, lane, E), axis=1, keepdims=True)
    oh2 = lane == i2

    eps = jnp.exp(m2 - m1)
    denom = 1.0 + eps
    w1 = 1.0 / denom
    w2 = eps / denom
    masks = jnp.where(oh1, w1, 0.0) + jnp.where(oh2, w2, 0.0)
    m_ref[...] = masks

    usage_part = jnp.sum(masks, axis=0, keepdims=True)
    cnt_part = jnp.sum((masks > 0.0).astype(jnp.float32))

    @pl.when(step == 0)
    def _init():
        u_ref[...] = jnp.zeros_like(u_ref)
        l_ref[...] = jnp.zeros_like(l_ref)

    u_ref[...] += usage_part
    l_ref[...] += jnp.reshape(cnt_part, (1, 1))

    @pl.when(step == nsteps - 1)
    def _finalize():
        u = u_ref[...]
        un = u / jnp.sum(u)
        u_ref[...] = un
        lbl = jnp.mean((un - 1.0 / E) ** 2)
        sparsity = l_ref[...] / (ntokens * K)
        l_ref[...] = lbl + 0.1 * sparsity


def kernel(x, W1, b1, W2, b2):
    B, S, _ = x.shape
    N = B * S
    nsteps = N // TM
    xf = x.reshape(N, H)

    grid_spec = pl.GridSpec(
        grid=(nsteps,),
        in_specs=[
            pl.BlockSpec((TM, H), lambda i: (i, 0)),
            pl.BlockSpec((H, 2 * H), lambda i: (0, 0)),
            pl.BlockSpec((1, 2 * H), lambda i: (0, 0)),
            pl.BlockSpec((2 * H, E), lambda i: (0, 0)),
            pl.BlockSpec((1, E), lambda i: (0, 0)),
        ],
        out_specs=[
            pl.BlockSpec((TM, E), lambda i: (i, 0)),
            pl.BlockSpec((TM, E), lambda i: (i, 0)),
            pl.BlockSpec((1, E), lambda i: (0, 0)),
            pl.BlockSpec((1, 1), lambda i: (0, 0)),
        ],
    )

    e_out, masks, usage, loss = pl.pallas_call(
        functools.partial(_router_body, nsteps, N),
        grid_spec=grid_spec,
        out_shape=[
            jax.ShapeDtypeStruct((N, E), jnp.float32),
            jax.ShapeDtypeStruct((N, E), jnp.float32),
            jax.ShapeDtypeStruct((1, E), jnp.float32),
            jax.ShapeDtypeStruct((1, 1), jnp.float32),
        ],
        compiler_params=pltpu.CompilerParams(
            dimension_semantics=("arbitrary",),
        ),
    )(xf, W1, b1.reshape(1, 2 * H), W2, b2.reshape(1, E))

    return (e_out.reshape(B, S, E), masks.reshape(B, S, E),
            loss[0, 0], usage[0])


# speedup vs baseline: 5.1339x; 1.0638x over previous
import functools

import jax
import jax.numpy as jnp
from jax.experimental import pallas as pl
from jax.experimental.pallas import tpu as pltpu

H = 768
E = 64
K = 2
TM = 4096


def _router_body(nsteps, ntokens, x_ref, w1_ref, b1_ref, w2_ref, b2_ref,
                 e_ref, m_ref, u_ref, l_ref):
    step = pl.program_id(0) * pl.num_programs(1) + pl.program_id(1)

    x = x_ref[0]
    h = jnp.dot(x, w1_ref[...], preferred_element_type=jnp.float32)
    h = h + b1_ref[...]
    h = 0.5 * h * (1.0 + jax.lax.erf(h * 0.7071067811865476))
    e = jnp.dot(h, w2_ref[...], preferred_element_type=jnp.float32)
    e = e + b2_ref[...]
    e_ref[0] = e

    lane = jax.lax.broadcasted_iota(jnp.int32, e.shape, 1)
    m1 = jnp.max(e, axis=1, keepdims=True)
    i1 = jnp.min(jnp.where(e == m1, lane, E), axis=1, keepdims=True)
    oh1 = lane == i1
    e2 = jnp.where(oh1, -jnp.inf, e)
    m2 = jnp.max(e2, axis=1, keepdims=True)
    i2 = jnp.min(jnp.where(e2 == m2, lane, E), axis=1, keepdims=True)
    oh2 = lane == i2

    eps = jnp.exp(m2 - m1)
    denom = 1.0 + eps
    w1 = 1.0 / denom
    w2 = eps / denom
    masks = jnp.where(oh1, w1, 0.0) + jnp.where(oh2, w2, 0.0)
    m_ref[0] = masks

    usage_part = jnp.sum(masks, axis=0, keepdims=True)
    cnt_part = jnp.sum((masks > 0.0).astype(jnp.float32))

    @pl.when(step == 0)
    def _init():
        u_ref[...] = jnp.zeros_like(u_ref)
        l_ref[...] = jnp.zeros_like(l_ref)

    u_ref[...] += usage_part
    l_ref[...] += jnp.reshape(cnt_part, (1, 1))

    @pl.when(step == nsteps - 1)
    def _finalize():
        u = u_ref[...]
        un = u / jnp.sum(u)
        u_ref[...] = un
        lbl = jnp.mean((un - 1.0 / E) ** 2)
        sparsity = l_ref[...] / (ntokens * K)
        l_ref[...] = lbl + 0.1 * sparsity


def kernel(x, W1, b1, W2, b2):
    B, S, _ = x.shape
    N = B * S
    nsteps = N // TM
    s_steps = S // TM

    grid_spec = pl.GridSpec(
        grid=(B, s_steps),
        in_specs=[
            pl.BlockSpec((1, TM, H), lambda b, s: (b, s, 0)),
            pl.BlockSpec((H, 2 * H), lambda b, s: (0, 0)),
            pl.BlockSpec((1, 2 * H), lambda b, s: (0, 0)),
            pl.BlockSpec((2 * H, E), lambda b, s: (0, 0)),
            pl.BlockSpec((1, E), lambda b, s: (0, 0)),
        ],
        out_specs=[
            pl.BlockSpec((1, TM, E), lambda b, s: (b, s, 0)),
            pl.BlockSpec((1, TM, E), lambda b, s: (b, s, 0)),
            pl.BlockSpec((1, E), lambda b, s: (0, 0)),
            pl.BlockSpec((1, 1), lambda b, s: (0, 0)),
        ],
    )

    e_out, masks, usage, loss = pl.pallas_call(
        functools.partial(_router_body, nsteps, N),
        grid_spec=grid_spec,
        out_shape=[
            jax.ShapeDtypeStruct((B, S, E), jnp.float32),
            jax.ShapeDtypeStruct((B, S, E), jnp.float32),
            jax.ShapeDtypeStruct((1, E), jnp.float32),
            jax.ShapeDtypeStruct((1, 1), jnp.float32),
        ],
        compiler_params=pltpu.CompilerParams(
            dimension_semantics=("arbitrary", "arbitrary"),
        ),
    )(x, W1, b1.reshape(1, 2 * H), W2, b2.reshape(1, E))

    return (e_out, masks, loss[0, 0], usage[0])
